# baseline (device time: 98472 ns/iter reference)
import jax
import jax.numpy as jnp
from jax import lax
from jax.experimental import pallas as pl
from jax.experimental.pallas import tpu as pltpu

N_DEV = 8
NSUB = 4
LANES = 2 * NSUB


def kernel(x, W1, W2):
    m, k = x.shape
    _, d = W1.shape
    _, f = W2.shape
    chunk = m // N_DEV
    lw = d // LANES

    lane_order = []
    for s in range(NSUB):
        lane_order += [s, NSUB + s]

    def body(x_ref, w1_ref, w2_ref, out_ref, part_ref,
             rs_buf, ag_buf, rs_s, rs_r, ag_s, ag_r):
        i = lax.axis_index("i")
        left = lax.rem(i - 1 + N_DEV, N_DEV)
        right = lax.rem(i + 1, N_DEV)

        barrier_sem = pltpu.get_barrier_semaphore()
        for nbr in (left, right):
            pl.semaphore_signal(
                barrier_sem, inc=1,
                device_id=(nbr,), device_id_type=pl.DeviceIdType.MESH,
            )
        pl.semaphore_wait(barrier_sem, 2)

        w1 = w1_ref[...].astype(jnp.bfloat16)
        w2l = [w2_ref[l * lw:(l + 1) * lw, :].astype(jnp.bfloat16)
               for l in range(LANES)]

        def rightward(l):
            return l < NSUB

        def dev(l):
            return right if rightward(l) else left

        def pchunk(c):
            xa = x_ref[pl.ds(c * chunk, chunk), :].astype(jnp.bfloat16)
            return jnp.dot(xa, w1, preferred_element_type=jnp.float32)

        def rdma(buf, sems_s, sems_r, l, src_slot, dst_slot, step):
            return pltpu.make_async_remote_copy(
                src_ref=buf.at[l, src_slot],
                dst_ref=buf.at[l, dst_slot],
                send_sem=sems_s.at[l, step],
                recv_sem=sems_r.at[l, step],
                device_id=(dev(l),), device_id_type=pl.DeviceIdType.MESH,
            )

        sent = []

        p7 = pchunk(lax.rem(i + 7, N_DEV))
        part_ref[7] = p7
        p1 = pchunk(lax.rem(i + 1, N_DEV))
        part_ref[1] = p1
        for l in lane_order:
            p = p7 if rightward(l) else p1
            rs_buf[l, 7] = p[:, l * lw:(l + 1) * lw].astype(jnp.bfloat16)
            r = rdma(rs_buf, rs_s, rs_r, l, 7, 0, 0)
            r.start()
            sent.append(r)

        for r_off in (6, 2, 5, 3, 4, 0):
            part_ref[r_off] = pchunk(lax.rem(i + r_off, N_DEV))

        acc = [None] * LANES
        for s in range(N_DEV - 1):
            for l in lane_order:
                rdma(rs_buf, rs_s, rs_r, l, s, s, s).wait_recv()
                r_off = (6 - s) if rightward(l) else (2 + s) % N_DEV
                acc[l] = (rs_buf[l, s].astype(jnp.float32)
                          + part_ref[r_off][:, l * lw:(l + 1) * lw])
                if s < N_DEV - 2:
                    rs_buf[l, s] = acc[l].astype(jnp.bfloat16)
                    r = rdma(rs_buf, rs_s, rs_r, l, s, s + 1, s + 1)
                    r.start()
                    sent.append(r)

        for l in lane_order:
            ag_buf[l, 7] = acc[l].astype(jnp.bfloat16)
            r = rdma(ag_buf, ag_s, ag_r, l, 7, 0, 0)
            r.start()
            sent.append(r)
        own = jnp.dot(ag_buf[0, 7], w2l[0], preferred_element_type=jnp.float32)
        for l in range(1, LANES):
            own = own + jnp.dot(ag_buf[l, 7], w2l[l],
                                preferred_element_type=jnp.float32)
        out_ref[pl.ds(i * chunk, chunk), :] = own

        for s in range(N_DEV - 1):
            for l in lane_order:
                rdma(ag_buf, ag_s, ag_r, l, s, s, s).wait_recv()
                if s < N_DEV - 2:
                    r = rdma(ag_buf, ag_s, ag_r, l, s, s + 1, s + 1)
                    r.start()
                    sent.append(r)
            pieceR = jnp.dot(ag_buf[0, s], w2l[0],
                             preferred_element_type=jnp.float32)
            for l in range(1, NSUB):
                pieceR = pieceR + jnp.dot(ag_buf[l, s], w2l[l],
                                          preferred_element_type=jnp.float32)
            pieceL = jnp.dot(ag_buf[NSUB, s], w2l[NSUB],
                             preferred_element_type=jnp.float32)
            for l in range(NSUB + 1, LANES):
                pieceL = pieceL + jnp.dot(ag_buf[l, s], w2l[l],
                                          preferred_element_type=jnp.float32)
            cR = lax.rem(i - 1 - s + 2 * N_DEV, N_DEV)
            cL = lax.rem(i + 1 + s, N_DEV)
            dsR = pl.ds(cR * chunk, chunk)
            dsL = pl.ds(cL * chunk, chunk)
            if s < 3:
                out_ref[dsR, :] = pieceR
                out_ref[dsL, :] = pieceL
            elif s == 3:
                out_ref[dsR, :] = pieceR + pieceL
            else:
                out_ref[dsR, :] = out_ref[dsR, :] + pieceR
                out_ref[dsL, :] = out_ref[dsL, :] + pieceL

        for r in sent:
            r.wait_send()

    return pl.pallas_call(
        body,
        out_shape=jax.ShapeDtypeStruct((m, f), jnp.float32),
        in_specs=[
            pl.BlockSpec(memory_space=pltpu.VMEM),
            pl.BlockSpec(memory_space=pltpu.VMEM),
            pl.BlockSpec(memory_space=pltpu.VMEM),
        ],
        out_specs=pl.BlockSpec(memory_space=pltpu.VMEM),
        scratch_shapes=[
            pltpu.VMEM((N_DEV, chunk, d), jnp.float32),
            pltpu.VMEM((LANES, N_DEV, chunk, lw), jnp.bfloat16),
            pltpu.VMEM((LANES, N_DEV, chunk, lw), jnp.bfloat16),
            pltpu.SemaphoreType.DMA((LANES, N_DEV - 1)),
            pltpu.SemaphoreType.DMA((LANES, N_DEV - 1)),
            pltpu.SemaphoreType.DMA((LANES, N_DEV - 1)),
            pltpu.SemaphoreType.DMA((LANES, N_DEV - 1)),
        ],
        compiler_params=pltpu.CompilerParams(collective_id=0),
    )(x, W1, W2)


# device time: 56248 ns/iter; 1.7507x vs baseline; 1.7507x over previous
import jax
import jax.numpy as jnp
from jax import lax
from jax.experimental import pallas as pl
from jax.experimental.pallas import tpu as pltpu

N_DEV = 8
P = 4
R, L = 0, 1


def kernel(x, W1, W2):
    m, k = x.shape
    _, d = W1.shape
    _, f = W2.shape
    chunk = m // N_DEV
    hd = d // 2
    rh = chunk // P

    def body(x_ref, w1_ref, w2_ref, out_ref, part_ref,
             rs_buf, ag_buf, rs_s, rs_r, ag_s, ag_r):
        i = lax.axis_index("i")
        left = lax.rem(i - 1 + N_DEV, N_DEV)
        right = lax.rem(i + 1, N_DEV)

        barrier_sem = pltpu.get_barrier_semaphore()
        for nbr in (left, right):
            pl.semaphore_signal(
                barrier_sem, inc=1,
                device_id=(nbr,), device_id_type=pl.DeviceIdType.MESH,
            )
        pl.semaphore_wait(barrier_sem, 2)

        w1 = w1_ref[...].astype(jnp.bfloat16)
        w2R = w2_ref[:hd, :].astype(jnp.bfloat16)
        w2L = w2_ref[hd:, :].astype(jnp.bfloat16)

        def pchunk(c):
            xa = x_ref[pl.ds(c * chunk, chunk), :].astype(jnp.bfloat16)
            return jnp.dot(xa, w1, preferred_element_type=jnp.float32)

        def cols(dirn):
            return slice(0, hd) if dirn == R else slice(hd, d)

        def rdma(buf, sems_s, sems_r, dirn, p, src_slot, dst_slot, step):
            rows = pl.ds(p * rh, rh)
            return pltpu.make_async_remote_copy(
                src_ref=buf.at[dirn, src_slot, rows],
                dst_ref=buf.at[dirn, dst_slot, rows],
                send_sem=sems_s.at[dirn, p, step],
                recv_sem=sems_r.at[dirn, p, step],
                device_id=(right if dirn == R else left,),
                device_id_type=pl.DeviceIdType.MESH,
            )

        sent = []

        p7 = pchunk(lax.rem(i + 7, N_DEV))
        part_ref[7] = p7
        rs_buf[R, 7] = p7[:, cols(R)].astype(jnp.bfloat16)
        for p in range(P):
            r = rdma(rs_buf, rs_s, rs_r, R, p, 7, 0, 0)
            r.start()
            sent.append(r)
        p1 = pchunk(lax.rem(i + 1, N_DEV))
        part_ref[1] = p1
        rs_buf[L, 7] = p1[:, cols(L)].astype(jnp.bfloat16)
        for p in range(P):
            r = rdma(rs_buf, rs_s, rs_r, L, p, 7, 0, 0)
            r.start()
            sent.append(r)

        for r_off in (6, 2, 5, 3, 4, 0):
            part_ref[r_off] = pchunk(lax.rem(i + r_off, N_DEV))

        accs = {}
        for s in range(N_DEV - 1):
            final = s == N_DEV - 2
            for p in range(P):
                for dirn in (R, L):
                    rdma(rs_buf, rs_s, rs_r, dirn, p, s, s, s).wait_recv()
                    r_off = (6 - s) if dirn == R else (2 + s) % N_DEV
                    rows = pl.ds(p * rh, rh)
                    acc = (rs_buf[dirn, s, rows].astype(jnp.float32)
                           + part_ref[r_off][p * rh:(p + 1) * rh, cols(dirn)])
                    if not final:
                        rs_buf[dirn, s, rows] = acc.astype(jnp.bfloat16)
                        r = rdma(rs_buf, rs_s, rs_r, dirn, p, s, s + 1, s + 1)
                        r.start()
                        sent.append(r)
                    else:
                        ag_buf[dirn, 7, rows] = acc.astype(jnp.bfloat16)
                        r = rdma(ag_buf, ag_s, ag_r, dirn, p, 7, 0, 0)
                        r.start()
                        sent.append(r)
                        accs[(dirn, p)] = acc
                if final:
                    out_ref[pl.ds(i * chunk + p * rh, rh), :] = (
                        jnp.dot(accs[(R, p)].astype(jnp.bfloat16), w2R,
                                preferred_element_type=jnp.float32)
                        + jnp.dot(accs[(L, p)].astype(jnp.bfloat16), w2L,
                                  preferred_element_type=jnp.float32)
                    )

        for s in range(N_DEV - 1):
            cR = lax.rem(i - 1 - s + 2 * N_DEV, N_DEV)
            cL = lax.rem(i + 1 + s, N_DEV)
            for p in range(P):
                for dirn in (R, L):
                    rdma(ag_buf, ag_s, ag_r, dirn, p, s, s, s).wait_recv()
                    if s < N_DEV - 2:
                        r = rdma(ag_buf, ag_s, ag_r, dirn, p, s, s + 1, s + 1)
                        r.start()
                        sent.append(r)
                rows = pl.ds(p * rh, rh)
                pieceR = jnp.dot(ag_buf[R, s, rows], w2R,
                                 preferred_element_type=jnp.float32)
                pieceL = jnp.dot(ag_buf[L, s, rows], w2L,
                                 preferred_element_type=jnp.float32)
                dR = pl.ds(cR * chunk + p * rh, rh)
                dL = pl.ds(cL * chunk + p * rh, rh)
                if s < 3:
                    out_ref[dR, :] = pieceR
                    out_ref[dL, :] = pieceL
                elif s == 3:
                    out_ref[dR, :] = pieceR + pieceL
                else:
                    out_ref[dR, :] = out_ref[dR, :] + pieceR
                    out_ref[dL, :] = out_ref[dL, :] + pieceL

        for r in sent:
            r.wait_send()

    return pl.pallas_call(
        body,
        out_shape=jax.ShapeDtypeStruct((m, f), jnp.float32),
        in_specs=[
            pl.BlockSpec(memory_space=pltpu.VMEM),
            pl.BlockSpec(memory_space=pltpu.VMEM),
            pl.BlockSpec(memory_space=pltpu.VMEM),
        ],
        out_specs=pl.BlockSpec(memory_space=pltpu.VMEM),
        scratch_shapes=[
            pltpu.VMEM((N_DEV, chunk, d), jnp.float32),
            pltpu.VMEM((2, N_DEV, chunk, hd), jnp.bfloat16),
            pltpu.VMEM((2, N_DEV, chunk, hd), jnp.bfloat16),
            pltpu.SemaphoreType.DMA((2, P, N_DEV - 1)),
            pltpu.SemaphoreType.DMA((2, P, N_DEV - 1)),
            pltpu.SemaphoreType.DMA((2, P, N_DEV - 1)),
            pltpu.SemaphoreType.DMA((2, P, N_DEV - 1)),
        ],
        compiler_params=pltpu.CompilerParams(collective_id=0),
    )(x, W1, W2)
